# Initial kernel scaffold; baseline (speedup 1.0000x reference)
import jax
import jax.numpy as jnp
from jax.experimental import pallas as pl


def kernel(x, emb):
    raise NotImplementedError("write your pallas kernel here")


# speedup vs baseline: 2.6574x; 2.6574x over previous
import jax
import jax.numpy as jnp
from jax.experimental import pallas as pl

_BT = 512


def _body(x_ref, e_ref, o_ref):
    o_ref[...] = x_ref[...] + e_ref[...]


def kernel(x, emb):
    B, T, D = x.shape
    nT = T // _BT
    return pl.pallas_call(
        _body,
        grid=(nT, B),
        in_specs=[
            pl.BlockSpec((1, _BT, D), lambda i, j: (j, i, 0)),
            pl.BlockSpec((_BT, D), lambda i, j: (i, 0)),
        ],
        out_specs=pl.BlockSpec((1, _BT, D), lambda i, j: (j, i, 0)),
        out_shape=jax.ShapeDtypeStruct((B, T, D), x.dtype),
    )(x, emb)
